# baseline (speedup 1.0000x reference)
import functools

import jax
import jax.numpy as jnp
from jax.experimental import pallas as pl
from jax.experimental.pallas import tpu as pltpu

_ALPHA = 0.1
_NG = 64


def _combine_body(agg_ref, h0_ref, w_ref, g_ref, be_ref, out_ref):
    h = (1.0 - _ALPHA) * agg_ref[...] + _ALPHA * h0_ref[...]
    hc = jnp.dot(h, w_ref[...], preferred_element_type=jnp.float32)
    hc = jnp.maximum(hc, 0.0)
    m = jnp.mean(hc, axis=0, keepdims=True)
    v = jnp.mean((hc - m) ** 2, axis=0, keepdims=True)
    out_ref[...] = g_ref[...] * (hc - m) / jnp.sqrt(v + 1e-5) + be_ref[...]


def _combine(agg, h0, w, g, be):
    n, hdim = agg.shape
    return pl.pallas_call(
        _combine_body,
        out_shape=jax.ShapeDtypeStruct((n, hdim), jnp.float32),
    )(agg, h0, w, g.reshape(1, hdim), be.reshape(1, hdim))


def _bn(x, g, b, eps=1e-5):
    m = jnp.mean(x, axis=0)
    v = jnp.var(x, axis=0)
    return g * (x - m) / jnp.sqrt(v + eps) + b


def _mlp_block(x, W, b, g, be):
    x1 = x @ W + b
    x2 = jax.nn.relu(x1)
    x2 = _bn(x2, g, be)
    return x2 + x1


def kernel(x, edge_attr, x_10d, lin_first, gcn_params, ewmlp_params, head_params, edge_index, batch):
    n = x.shape[0]
    ew = edge_attr
    for (W, b, g, be) in ewmlp_params:
        ew = _mlp_block(ew, W, b, g, be)
    ew = jax.nn.sigmoid(ew)[:, 0]
    Wf, bf = lin_first
    h = x @ Wf + bf
    h0 = h
    row = edge_index[0]
    col = edge_index[1]
    deg = jax.ops.segment_sum(ew, col, num_segments=n) + 1.0
    dis = 1.0 / jnp.sqrt(deg)
    norm = dis[row] * ew * dis[col]
    inv_deg = dis * dis
    for (W1, g, be) in gcn_params:
        msg = h[row] * norm[:, None]
        agg = jax.ops.segment_sum(msg, col, num_segments=n)
        agg = agg + h * inv_deg[:, None]
        h = _combine(agg, h0, W1, g, be)
    x_aggr = jax.ops.segment_sum(h, batch, num_segments=_NG)
    x_aggr = jax.nn.sigmoid(x_aggr)
    x_aggr = jnp.concatenate([x_aggr, x_10d], axis=1)
    out = x_aggr
    for (W, b, g, be) in head_params:
        out = _mlp_block(out, W, b, g, be)
    out = jax.nn.sigmoid(out)
    return (out, x_aggr)


# speedup vs baseline: 1.3484x; 1.3484x over previous
import functools

import jax
import jax.numpy as jnp
from jax import lax
from jax.experimental import pallas as pl
from jax.experimental.pallas import tpu as pltpu
from jax.experimental.pallas import tpu_sc as plsc

_ALPHA = 0.1
_NG = 64
_N = 10000
_H = 128
_CH = 128
_NCHUNK = 320
_GRP = 16
_NGRP = _NCHUNK // _GRP
_EPT = _CH * _NCHUNK
_NSHARD = 16
_NPAD = 10240
_RPT = _NPAD // 16


def _spmm_kernel(h_hbm, row_hbm, col_hbm, norm_hbm, out_hbm,
                 rowv, colv, normv, buf, acc, gsem):
    s = lax.axis_index("s")
    def _zrow(i, carry):
        for j in range(_H // 16):
            buf[i, pl.ds(j * 16, 16)] = jnp.zeros((16,), jnp.float32)
        return carry
    lax.fori_loop(0, _CH, _zrow, 0)
    base = s * _RPT
    for t in range(_RPT // _CH):
        pltpu.sync_copy(buf, acc.at[pl.ds(base + t * _CH, _CH)])
    plsc.subcore_barrier()

    def _group(g, carry):
        pltpu.sync_copy(row_hbm.at[s, pl.ds(g * _GRP, _GRP)], rowv)
        pltpu.sync_copy(col_hbm.at[s, pl.ds(g * _GRP, _GRP)], colv)
        pltpu.sync_copy(norm_hbm.at[s, pl.ds(g * _GRP, _GRP)], normv)

        def _chunk(i, c1):
            pltpu.async_copy(h_hbm.at[rowv.at[i]], buf, gsem).wait()
            def _escale(eb, c2):
                e0 = eb * 16
                n16 = normv[i, pl.ds(e0, 16)]
                for j in range(16):
                    sc = n16[j]
                    for k in range(_H // 16):
                        sl = pl.ds(k * 16, 16)
                        buf[e0 + j, sl] = buf[e0 + j, sl] * sc
                return c2
            lax.fori_loop(0, _CH // 16, _escale, 0)
            pltpu.sync_copy(buf, acc.at[colv.at[i]], add=True)
            return c1
        lax.fori_loop(0, _GRP, _chunk, 0)
        return carry
    lax.fori_loop(0, _NGRP, _group, 0)
    plsc.subcore_barrier()
    for t in range(_RPT // _CH):
        pltpu.sync_copy(acc.at[pl.ds(base + t * _CH, _CH)],
                        out_hbm.at[pl.ds(base + t * _CH, _CH)])


def _spmm(h, row_p, col_p, norm_p):
    mesh = plsc.VectorSubcoreMesh(core_axis_name="c", subcore_axis_name="s",
                                  num_cores=1)
    f = pl.kernel(
        _spmm_kernel,
        out_type=jax.ShapeDtypeStruct((_NPAD, _H), jnp.float32),
        mesh=mesh,
        scratch_types=[
            pltpu.VMEM((_GRP, _CH), jnp.int32),
            pltpu.VMEM((_GRP, _CH), jnp.int32),
            pltpu.VMEM((_GRP, _CH), jnp.float32),
            pltpu.VMEM((_CH, _H), jnp.float32),
            pltpu.VMEM_SHARED((_NPAD, _H), jnp.float32),
            pltpu.SemaphoreType.DMA,
        ],
    )
    return f(h, row_p, col_p, norm_p)


def _combine_body(p_ref, h_ref, h0_ref, dis_ref, w_ref, g_ref, be_ref, out_ref):
    inv_deg = dis_ref[...] * dis_ref[...]
    agg = p_ref[:_N, :] + h_ref[...] * inv_deg
    hcomb = (1.0 - _ALPHA) * agg + _ALPHA * h0_ref[...]
    hc = jnp.dot(hcomb, w_ref[...], preferred_element_type=jnp.float32)
    hc = jnp.maximum(hc, 0.0)
    m = jnp.mean(hc, axis=0, keepdims=True)
    v = jnp.mean((hc - m) ** 2, axis=0, keepdims=True)
    out_ref[...] = g_ref[...] * (hc - m) / jnp.sqrt(v + 1e-5) + be_ref[...]


def _combine(p, h, h0, dis, w, g, be):
    return pl.pallas_call(
        _combine_body,
        out_shape=jax.ShapeDtypeStruct((_N, _H), jnp.float32),
    )(p, h, h0, dis.reshape(_N, 1), w, g.reshape(1, _H), be.reshape(1, _H))


def _bn(x, g, b, eps=1e-5):
    m = jnp.mean(x, axis=0)
    v = jnp.var(x, axis=0)
    return g * (x - m) / jnp.sqrt(v + eps) + b


def _mlp_block(x, W, b, g, be):
    x1 = x @ W + b
    x2 = jax.nn.relu(x1)
    x2 = _bn(x2, g, be)
    return x2 + x1


def kernel(x, edge_attr, x_10d, lin_first, gcn_params, ewmlp_params, head_params, edge_index, batch):
    ew = edge_attr
    for (W, b, g, be) in ewmlp_params:
        ew = _mlp_block(ew, W, b, g, be)
    ew = jax.nn.sigmoid(ew)[:, 0]
    Wf, bf = lin_first
    h = x @ Wf + bf
    h0 = h
    row = edge_index[0]
    col = edge_index[1]
    deg = jax.ops.segment_sum(ew, col, num_segments=_N) + 1.0
    dis = 1.0 / jnp.sqrt(deg)
    norm = dis[row] * ew * dis[col]
    pad = ((0, 0), (0, _EPT - 640000 // _NSHARD))
    row_p = jnp.pad(row.reshape(_NSHARD, -1), pad).reshape(_NSHARD, _NCHUNK, _CH)
    col_p = jnp.pad(col.reshape(_NSHARD, -1), pad).reshape(_NSHARD, _NCHUNK, _CH)
    norm_p = jnp.pad(norm.reshape(_NSHARD, -1), pad).reshape(_NSHARD, _NCHUNK, _CH)
    for (W1, g, be) in gcn_params:
        p = _spmm(h, row_p, col_p, norm_p)
        h = _combine(p, h, h0, dis, W1, g, be)
    x_aggr = jax.ops.segment_sum(h, batch, num_segments=_NG)
    x_aggr = jax.nn.sigmoid(x_aggr)
    x_aggr = jnp.concatenate([x_aggr, x_10d], axis=1)
    out = x_aggr
    for (W, b, g, be) in head_params:
        out = _mlp_block(out, W, b, g, be)
    out = jax.nn.sigmoid(out)
    return (out, x_aggr)


# speedup vs baseline: 4.7915x; 3.5535x over previous
import functools

import jax
import jax.numpy as jnp
from jax import lax
from jax.experimental import pallas as pl
from jax.experimental.pallas import tpu as pltpu
from jax.experimental.pallas import tpu_sc as plsc

_ALPHA = 0.1
_NG = 64
_N = 10000
_H = 128
_CH = 128
_NCHUNK = 320
_GRP = 16
_NGRP = _NCHUNK // _GRP
_EPT = _CH * _NCHUNK
_NSHARD = 16
_NPAD = 10240
_RPT = _NPAD // 16


def _spmm_kernel(h_hbm, row_hbm, col_hbm, norm_hbm, out_hbm,
                 rowv, colv, normv, buf, acc, gsem):
    s = lax.axis_index("s")
    def _zrow(i, carry):
        for j in range(_H // 16):
            buf[i, pl.ds(j * 16, 16)] = jnp.zeros((16,), jnp.float32)
        return carry
    lax.fori_loop(0, _CH, _zrow, 0)
    base = s * _RPT
    for t in range(_RPT // _CH):
        pltpu.sync_copy(buf, acc.at[pl.ds(base + t * _CH, _CH)])
    plsc.subcore_barrier()

    def _group(g, carry):
        pltpu.sync_copy(row_hbm.at[s, pl.ds(g * _GRP, _GRP)], rowv)
        pltpu.sync_copy(col_hbm.at[s, pl.ds(g * _GRP, _GRP)], colv)
        pltpu.sync_copy(norm_hbm.at[s, pl.ds(g * _GRP, _GRP)], normv)

        def _chunk(i, c1):
            pltpu.async_copy(h_hbm.at[rowv.at[i]], buf, gsem).wait()
            def _escale(eb, c2):
                e0 = eb * 16
                n16 = normv[i, pl.ds(e0, 16)]
                for j in range(16):
                    sc = n16[j]
                    for k in range(_H // 16):
                        sl = pl.ds(k * 16, 16)
                        buf[e0 + j, sl] = buf[e0 + j, sl] * sc
                return c2
            lax.fori_loop(0, _CH // 16, _escale, 0)
            pltpu.sync_copy(buf, acc.at[colv.at[i]], add=True)
            return c1
        lax.fori_loop(0, _GRP, _chunk, 0)
        return carry
    lax.fori_loop(0, _NGRP, _group, 0)
    plsc.subcore_barrier()
    for t in range(_RPT // _CH):
        pltpu.sync_copy(acc.at[pl.ds(base + t * _CH, _CH)],
                        out_hbm.at[pl.ds(base + t * _CH, _CH)])


def _spmm(h, row_p, col_p, norm_p):
    mesh = plsc.VectorSubcoreMesh(core_axis_name="c", subcore_axis_name="s",
                                  num_cores=1)
    f = pl.kernel(
        _spmm_kernel,
        out_type=jax.ShapeDtypeStruct((_NPAD, _H), jnp.float32),
        mesh=mesh,
        scratch_types=[
            pltpu.VMEM((_GRP, _CH), jnp.int32),
            pltpu.VMEM((_GRP, _CH), jnp.int32),
            pltpu.VMEM((_GRP, _CH), jnp.float32),
            pltpu.VMEM((_CH, _H), jnp.float32),
            pltpu.VMEM_SHARED((_NPAD, _H), jnp.float32),
            pltpu.SemaphoreType.DMA,
        ],
    )
    return f(h, row_p, col_p, norm_p)


def _norm_kernel(row_hbm, col_hbm, ew_hbm, norm_hbm, dis_hbm,
                 rowv, colv, ewv, normb, zbuf, disv, degacc, gsem):
    s = lax.axis_index("s")
    def _z(i, carry):
        zbuf[pl.ds(i * 16, 16)] = jnp.zeros((16,), jnp.float32)
        return carry
    lax.fori_loop(0, _RPT // 16, _z, 0)
    pltpu.sync_copy(zbuf, degacc.at[pl.ds(s * _RPT, _RPT)])
    plsc.subcore_barrier()

    def _dgroup(g, carry):
        pltpu.sync_copy(col_hbm.at[s, pl.ds(g * _GRP, _GRP)], colv)
        pltpu.sync_copy(ew_hbm.at[s, pl.ds(g * _GRP, _GRP)], ewv)
        def _dchunk(i, c1):
            pltpu.sync_copy(ewv.at[i], degacc.at[colv.at[i]], add=True)
            return c1
        lax.fori_loop(0, _GRP, _dchunk, 0)
        return carry
    lax.fori_loop(0, _NGRP, _dgroup, 0)
    plsc.subcore_barrier()

    pltpu.sync_copy(degacc, disv)
    def _rsq(k, carry):
        sl = pl.ds(k * 16, 16)
        xv = disv[sl] + 1.0
        iv = 0x5F3759DF - lax.shift_right_logical(plsc.bitcast(xv, jnp.int32), 1)
        y = plsc.bitcast(iv, jnp.float32)
        xh = 0.5 * xv
        for _ in range(3):
            y = y * (1.5 - xh * y * y)
        disv[sl] = y
        return carry
    lax.fori_loop(0, _NPAD // 16, _rsq, 0)

    def _ngroup(g, carry):
        pltpu.sync_copy(row_hbm.at[s, pl.ds(g * _GRP, _GRP)], rowv)
        pltpu.sync_copy(col_hbm.at[s, pl.ds(g * _GRP, _GRP)], colv)
        pltpu.sync_copy(ew_hbm.at[s, pl.ds(g * _GRP, _GRP)], ewv)
        def _nchunk(i, c1):
            for eb in range(_CH // 16):
                sl = pl.ds(eb * 16, 16)
                r16 = rowv[i, sl]
                c16 = colv[i, sl]
                e16 = ewv[i, sl]
                dr = plsc.load_gather(disv, [r16])
                dc = plsc.load_gather(disv, [c16])
                normb[i, sl] = dr * e16 * dc
            return c1
        lax.fori_loop(0, _GRP, _nchunk, 0)
        pltpu.sync_copy(normb, norm_hbm.at[s, pl.ds(g * _GRP, _GRP)])
        return carry
    lax.fori_loop(0, _NGRP, _ngroup, 0)
    @pl.when(s == 0)
    def _():
        pltpu.sync_copy(disv, dis_hbm)


def _norm_sc(row_p, col_p, ew_p):
    mesh = plsc.VectorSubcoreMesh(core_axis_name="c", subcore_axis_name="s",
                                  num_cores=1)
    f = pl.kernel(
        _norm_kernel,
        out_type=(jax.ShapeDtypeStruct((_NSHARD, _NCHUNK, _CH), jnp.float32),
                  jax.ShapeDtypeStruct((_NPAD,), jnp.float32)),
        mesh=mesh,
        scratch_types=[
            pltpu.VMEM((_GRP, _CH), jnp.int32),
            pltpu.VMEM((_GRP, _CH), jnp.int32),
            pltpu.VMEM((_GRP, _CH), jnp.float32),
            pltpu.VMEM((_GRP, _CH), jnp.float32),
            pltpu.VMEM((_RPT,), jnp.float32),
            pltpu.VMEM((_NPAD,), jnp.float32),
            pltpu.VMEM_SHARED((_NPAD,), jnp.float32),
            pltpu.SemaphoreType.DMA,
        ],
        compiler_params=pltpu.CompilerParams(needs_layout_passes=False),
    )
    return f(row_p, col_p, ew_p)


def _combine_body(p_ref, h_ref, h0_ref, dis_ref, w_ref, g_ref, be_ref, out_ref):
    inv_deg = dis_ref[...] * dis_ref[...]
    agg = p_ref[:_N, :] + h_ref[...] * inv_deg
    hcomb = (1.0 - _ALPHA) * agg + _ALPHA * h0_ref[...]
    hc = jnp.dot(hcomb, w_ref[...], preferred_element_type=jnp.float32)
    hc = jnp.maximum(hc, 0.0)
    m = jnp.mean(hc, axis=0, keepdims=True)
    v = jnp.mean((hc - m) ** 2, axis=0, keepdims=True)
    out_ref[...] = g_ref[...] * (hc - m) / jnp.sqrt(v + 1e-5) + be_ref[...]


def _combine(p, h, h0, dis, w, g, be):
    return pl.pallas_call(
        _combine_body,
        out_shape=jax.ShapeDtypeStruct((_N, _H), jnp.float32),
    )(p, h, h0, dis.reshape(_N, 1), w, g.reshape(1, _H), be.reshape(1, _H))


def _bn(x, g, b, eps=1e-5):
    m = jnp.mean(x, axis=0)
    v = jnp.var(x, axis=0)
    return g * (x - m) / jnp.sqrt(v + eps) + b


def _mlp_block(x, W, b, g, be):
    x1 = x @ W + b
    x2 = jax.nn.relu(x1)
    x2 = _bn(x2, g, be)
    return x2 + x1


def kernel(x, edge_attr, x_10d, lin_first, gcn_params, ewmlp_params, head_params, edge_index, batch):
    ew = edge_attr
    for (W, b, g, be) in ewmlp_params:
        ew = _mlp_block(ew, W, b, g, be)
    ew = jax.nn.sigmoid(ew)[:, 0]
    Wf, bf = lin_first
    h = x @ Wf + bf
    h0 = h
    pad = ((0, 0), (0, _EPT - 640000 // _NSHARD))
    row_p = jnp.pad(edge_index[0].reshape(_NSHARD, -1), pad).reshape(_NSHARD, _NCHUNK, _CH)
    col_p = jnp.pad(edge_index[1].reshape(_NSHARD, -1), pad).reshape(_NSHARD, _NCHUNK, _CH)
    ew_p = jnp.pad(ew.reshape(_NSHARD, -1), pad).reshape(_NSHARD, _NCHUNK, _CH)
    norm_p, dis_pad = _norm_sc(row_p, col_p, ew_p)
    dis = dis_pad[:_N]
    for (W1, g, be) in gcn_params:
        p = _spmm(h, row_p, col_p, norm_p)
        h = _combine(p, h, h0, dis, W1, g, be)
    x_aggr = jax.ops.segment_sum(h, batch, num_segments=_NG)
    x_aggr = jax.nn.sigmoid(x_aggr)
    x_aggr = jnp.concatenate([x_aggr, x_10d], axis=1)
    out = x_aggr
    for (W, b, g, be) in head_params:
        out = _mlp_block(out, W, b, g, be)
    out = jax.nn.sigmoid(out)
    return (out, x_aggr)


# speedup vs baseline: 6.0677x; 1.2664x over previous
import functools

import jax
import jax.numpy as jnp
from jax import lax
from jax.experimental import pallas as pl
from jax.experimental.pallas import tpu as pltpu
from jax.experimental.pallas import tpu_sc as plsc

_ALPHA = 0.1
_NG = 64
_N = 10000
_H = 128
_CH = 128
_NCHUNK = 320
_GRP = 16
_NGRP = _NCHUNK // _GRP
_EPT = _CH * _NCHUNK
_NSHARD = 16
_NPAD = 10240
_RPT = _NPAD // 16


def _scale_chunk(buf, normv, i):
    def _escale(eb, c2):
        e0 = eb * 16
        n16 = normv[i, pl.ds(e0, 16)]
        for j in range(16):
            sc = n16[j]
            for k in range(_H // 16):
                sl = pl.ds(k * 16, 16)
                buf[e0 + j, sl] = buf[e0 + j, sl] * sc
        return c2
    lax.fori_loop(0, _CH // 16, _escale, 0)


def _spmm_kernel(h_hbm, row_hbm, col_hbm, norm_hbm, out_hbm,
                 rowv, colv, normv, bufa, bufb, acc, sema, semb):
    s = lax.axis_index("s")
    def _zrow(i, carry):
        for j in range(_H // 16):
            bufa[i, pl.ds(j * 16, 16)] = jnp.zeros((16,), jnp.float32)
        return carry
    lax.fori_loop(0, _CH, _zrow, 0)
    base = s * _RPT
    for t in range(_RPT // _CH):
        pltpu.sync_copy(bufa, acc.at[pl.ds(base + t * _CH, _CH)])
    plsc.subcore_barrier()

    def _group(g, carry):
        pltpu.sync_copy(row_hbm.at[s, pl.ds(g * _GRP, _GRP)], rowv)
        pltpu.sync_copy(col_hbm.at[s, pl.ds(g * _GRP, _GRP)], colv)
        pltpu.sync_copy(norm_hbm.at[s, pl.ds(g * _GRP, _GRP)], normv)
        pltpu.async_copy(h_hbm.at[rowv.at[0]], bufa, sema)

        def _pair(k, c1):
            i0 = 2 * k
            i1 = i0 + 1
            pltpu.async_copy(h_hbm.at[rowv.at[i1]], bufb, semb)
            pltpu.make_async_copy(h_hbm.at[rowv.at[i0]], bufa, sema).wait()
            _scale_chunk(bufa, normv, i0)
            pltpu.sync_copy(bufa, acc.at[colv.at[i0]], add=True)
            @pl.when(k < _GRP // 2 - 1)
            def _():
                pltpu.async_copy(h_hbm.at[rowv.at[i0 + 2]], bufa, sema)
            pltpu.make_async_copy(h_hbm.at[rowv.at[i1]], bufb, semb).wait()
            _scale_chunk(bufb, normv, i1)
            pltpu.sync_copy(bufb, acc.at[colv.at[i1]], add=True)
            return c1
        lax.fori_loop(0, _GRP // 2, _pair, 0)
        return carry
    lax.fori_loop(0, _NGRP, _group, 0)
    plsc.subcore_barrier()
    for t in range(_RPT // _CH):
        pltpu.sync_copy(acc.at[pl.ds(base + t * _CH, _CH)],
                        out_hbm.at[pl.ds(base + t * _CH, _CH)])


def _spmm(h, row_p, col_p, norm_p):
    mesh = plsc.VectorSubcoreMesh(core_axis_name="c", subcore_axis_name="s",
                                  num_cores=1)
    f = pl.kernel(
        _spmm_kernel,
        out_type=jax.ShapeDtypeStruct((_NPAD, _H), jnp.float32),
        mesh=mesh,
        scratch_types=[
            pltpu.VMEM((_GRP, _CH), jnp.int32),
            pltpu.VMEM((_GRP, _CH), jnp.int32),
            pltpu.VMEM((_GRP, _CH), jnp.float32),
            pltpu.VMEM((_CH, _H), jnp.float32),
            pltpu.VMEM((_CH, _H), jnp.float32),
            pltpu.VMEM_SHARED((_NPAD, _H), jnp.float32),
            pltpu.SemaphoreType.DMA,
            pltpu.SemaphoreType.DMA,
        ],
        compiler_params=pltpu.CompilerParams(needs_layout_passes=False),
    )
    return f(h, row_p, col_p, norm_p)


def _norm_kernel(row_hbm, col_hbm, ew_hbm, norm_hbm, dis_hbm,
                 rowv, colv, ewv, normb, zbuf, disv, degacc, gsem):
    s = lax.axis_index("s")
    def _z(i, carry):
        zbuf[pl.ds(i * 16, 16)] = jnp.zeros((16,), jnp.float32)
        return carry
    lax.fori_loop(0, _RPT // 16, _z, 0)
    pltpu.sync_copy(zbuf, degacc.at[pl.ds(s * _RPT, _RPT)])
    plsc.subcore_barrier()

    def _dgroup(g, carry):
        pltpu.sync_copy(col_hbm.at[s, pl.ds(g * _GRP, _GRP)], colv)
        pltpu.sync_copy(ew_hbm.at[s, pl.ds(g * _GRP, _GRP)], ewv)
        def _dchunk(i, c1):
            pltpu.sync_copy(ewv.at[i], degacc.at[colv.at[i]], add=True)
            return c1
        lax.fori_loop(0, _GRP, _dchunk, 0)
        return carry
    lax.fori_loop(0, _NGRP, _dgroup, 0)
    plsc.subcore_barrier()

    pltpu.sync_copy(degacc, disv)
    def _rsq(k, carry):
        sl = pl.ds(k * 16, 16)
        xv = disv[sl] + 1.0
        iv = 0x5F3759DF - lax.shift_right_logical(plsc.bitcast(xv, jnp.int32), 1)
        y = plsc.bitcast(iv, jnp.float32)
        xh = 0.5 * xv
        for _ in range(3):
            y = y * (1.5 - xh * y * y)
        disv[sl] = y
        return carry
    lax.fori_loop(0, _NPAD // 16, _rsq, 0)

    def _ngroup(g, carry):
        pltpu.sync_copy(row_hbm.at[s, pl.ds(g * _GRP, _GRP)], rowv)
        pltpu.sync_copy(col_hbm.at[s, pl.ds(g * _GRP, _GRP)], colv)
        pltpu.sync_copy(ew_hbm.at[s, pl.ds(g * _GRP, _GRP)], ewv)
        def _nchunk(i, c1):
            for eb in range(_CH // 16):
                sl = pl.ds(eb * 16, 16)
                r16 = rowv[i, sl]
                c16 = colv[i, sl]
                e16 = ewv[i, sl]
                dr = plsc.load_gather(disv, [r16])
                dc = plsc.load_gather(disv, [c16])
                normb[i, sl] = dr * e16 * dc
            return c1
        lax.fori_loop(0, _GRP, _nchunk, 0)
        pltpu.sync_copy(normb, norm_hbm.at[s, pl.ds(g * _GRP, _GRP)])
        return carry
    lax.fori_loop(0, _NGRP, _ngroup, 0)
    @pl.when(s == 0)
    def _():
        pltpu.sync_copy(disv, dis_hbm)


def _norm_sc(row_p, col_p, ew_p):
    mesh = plsc.VectorSubcoreMesh(core_axis_name="c", subcore_axis_name="s",
                                  num_cores=1)
    f = pl.kernel(
        _norm_kernel,
        out_type=(jax.ShapeDtypeStruct((_NSHARD, _NCHUNK, _CH), jnp.float32),
                  jax.ShapeDtypeStruct((_NPAD,), jnp.float32)),
        mesh=mesh,
        scratch_types=[
            pltpu.VMEM((_GRP, _CH), jnp.int32),
            pltpu.VMEM((_GRP, _CH), jnp.int32),
            pltpu.VMEM((_GRP, _CH), jnp.float32),
            pltpu.VMEM((_GRP, _CH), jnp.float32),
            pltpu.VMEM((_RPT,), jnp.float32),
            pltpu.VMEM((_NPAD,), jnp.float32),
            pltpu.VMEM_SHARED((_NPAD,), jnp.float32),
            pltpu.SemaphoreType.DMA,
        ],
        compiler_params=pltpu.CompilerParams(needs_layout_passes=False),
    )
    return f(row_p, col_p, ew_p)


def _combine_body(p_ref, h_ref, h0_ref, dis_ref, w_ref, g_ref, be_ref, out_ref):
    inv_deg = dis_ref[...] * dis_ref[...]
    agg = p_ref[:_N, :] + h_ref[...] * inv_deg
    hcomb = (1.0 - _ALPHA) * agg + _ALPHA * h0_ref[...]
    hc = jnp.dot(hcomb, w_ref[...], preferred_element_type=jnp.float32)
    hc = jnp.maximum(hc, 0.0)
    m = jnp.mean(hc, axis=0, keepdims=True)
    v = jnp.mean((hc - m) ** 2, axis=0, keepdims=True)
    out_ref[...] = g_ref[...] * (hc - m) / jnp.sqrt(v + 1e-5) + be_ref[...]


def _combine(p, h, h0, dis, w, g, be):
    return pl.pallas_call(
        _combine_body,
        out_shape=jax.ShapeDtypeStruct((_N, _H), jnp.float32),
    )(p, h, h0, dis.reshape(_N, 1), w, g.reshape(1, _H), be.reshape(1, _H))


def _bn(x, g, b, eps=1e-5):
    m = jnp.mean(x, axis=0)
    v = jnp.var(x, axis=0)
    return g * (x - m) / jnp.sqrt(v + eps) + b


def _mlp_block(x, W, b, g, be):
    x1 = x @ W + b
    x2 = jax.nn.relu(x1)
    x2 = _bn(x2, g, be)
    return x2 + x1


def kernel(x, edge_attr, x_10d, lin_first, gcn_params, ewmlp_params, head_params, edge_index, batch):
    ew = edge_attr
    for (W, b, g, be) in ewmlp_params:
        ew = _mlp_block(ew, W, b, g, be)
    ew = jax.nn.sigmoid(ew)[:, 0]
    Wf, bf = lin_first
    h = x @ Wf + bf
    h0 = h
    pad = ((0, 0), (0, _EPT - 640000 // _NSHARD))
    row_p = jnp.pad(edge_index[0].reshape(_NSHARD, -1), pad).reshape(_NSHARD, _NCHUNK, _CH)
    col_p = jnp.pad(edge_index[1].reshape(_NSHARD, -1), pad).reshape(_NSHARD, _NCHUNK, _CH)
    ew_p = jnp.pad(ew.reshape(_NSHARD, -1), pad).reshape(_NSHARD, _NCHUNK, _CH)
    norm_p, dis_pad = _norm_sc(row_p, col_p, ew_p)
    dis = dis_pad[:_N]
    for (W1, g, be) in gcn_params:
        p = _spmm(h, row_p, col_p, norm_p)
        h = _combine(p, h, h0, dis, W1, g, be)
    x_aggr = jax.ops.segment_sum(h, batch, num_segments=_NG)
    x_aggr = jax.nn.sigmoid(x_aggr)
    x_aggr = jnp.concatenate([x_aggr, x_10d], axis=1)
    out = x_aggr
    for (W, b, g, be) in head_params:
        out = _mlp_block(out, W, b, g, be)
    out = jax.nn.sigmoid(out)
    return (out, x_aggr)


# speedup vs baseline: 7.7074x; 1.2702x over previous
import functools

import jax
import jax.numpy as jnp
from jax import lax
from jax.experimental import pallas as pl
from jax.experimental.pallas import tpu as pltpu
from jax.experimental.pallas import tpu_sc as plsc

_ALPHA = 0.1
_NG = 64
_N = 10000
_H = 128
_CH = 128
_NCHUNK = 160
_GRP = 16
_NGRP = _NCHUNK // _GRP
_EPT = _CH * _NCHUNK
_NSHARD = 32
_NPAD = 10240
_RPT = _NPAD // 16


def _scale_chunk(buf, normv, i):
    def _escale(eb, c2):
        e0 = eb * 16
        n16 = normv[i, pl.ds(e0, 16)]
        for j in range(16):
            sc = n16[j]
            for k in range(_H // 16):
                sl = pl.ds(k * 16, 16)
                buf[e0 + j, sl] = buf[e0 + j, sl] * sc
        return c2
    lax.fori_loop(0, _CH // 16, _escale, 0)


def _spmm_kernel(h_hbm, row_hbm, col_hbm, norm_hbm, out_hbm,
                 rowv, colv, normv, bufa, bufb, acc, sema, semb):
    c = lax.axis_index("c")
    s = lax.axis_index("s")
    wid = c * 16 + s
    def _zrow(i, carry):
        for j in range(_H // 16):
            bufa[i, pl.ds(j * 16, 16)] = jnp.zeros((16,), jnp.float32)
        return carry
    lax.fori_loop(0, _CH, _zrow, 0)
    base = s * _RPT
    for t in range(_RPT // _CH):
        pltpu.sync_copy(bufa, acc.at[pl.ds(base + t * _CH, _CH)])
    plsc.subcore_barrier()

    def _group(g, carry):
        pltpu.sync_copy(row_hbm.at[wid, pl.ds(g * _GRP, _GRP)], rowv)
        pltpu.sync_copy(col_hbm.at[wid, pl.ds(g * _GRP, _GRP)], colv)
        pltpu.sync_copy(norm_hbm.at[wid, pl.ds(g * _GRP, _GRP)], normv)
        pltpu.async_copy(h_hbm.at[rowv.at[0]], bufa, sema)

        def _pair(k, c1):
            i0 = 2 * k
            i1 = i0 + 1
            pltpu.async_copy(h_hbm.at[rowv.at[i1]], bufb, semb)
            pltpu.make_async_copy(h_hbm.at[rowv.at[i0]], bufa, sema).wait()
            _scale_chunk(bufa, normv, i0)
            pltpu.sync_copy(bufa, acc.at[colv.at[i0]], add=True)
            @pl.when(k < _GRP // 2 - 1)
            def _():
                pltpu.async_copy(h_hbm.at[rowv.at[i0 + 2]], bufa, sema)
            pltpu.make_async_copy(h_hbm.at[rowv.at[i1]], bufb, semb).wait()
            _scale_chunk(bufb, normv, i1)
            pltpu.sync_copy(bufb, acc.at[colv.at[i1]], add=True)
            return c1
        lax.fori_loop(0, _GRP // 2, _pair, 0)
        return carry
    lax.fori_loop(0, _NGRP, _group, 0)
    plsc.subcore_barrier()
    for t in range(_RPT // _CH):
        pltpu.sync_copy(acc.at[pl.ds(base + t * _CH, _CH)],
                        out_hbm.at[c, pl.ds(base + t * _CH, _CH)])


def _spmm(h, row_p, col_p, norm_p):
    mesh = plsc.VectorSubcoreMesh(core_axis_name="c", subcore_axis_name="s")
    f = pl.kernel(
        _spmm_kernel,
        out_type=jax.ShapeDtypeStruct((2, _NPAD, _H), jnp.float32),
        mesh=mesh,
        scratch_types=[
            pltpu.VMEM((_GRP, _CH), jnp.int32),
            pltpu.VMEM((_GRP, _CH), jnp.int32),
            pltpu.VMEM((_GRP, _CH), jnp.float32),
            pltpu.VMEM((_CH, _H), jnp.float32),
            pltpu.VMEM((_CH, _H), jnp.float32),
            pltpu.VMEM_SHARED((_NPAD, _H), jnp.float32),
            pltpu.SemaphoreType.DMA,
            pltpu.SemaphoreType.DMA,
        ],
        compiler_params=pltpu.CompilerParams(needs_layout_passes=False),
    )
    return f(h, row_p, col_p, norm_p)


def _norm_kernel(row_hbm, col_hbm, ew_hbm, norm_hbm, dis_hbm,
                 rowv, colv, ewv, normb, zbuf, disv, degacc, gsem):
    c = lax.axis_index("c")
    s = lax.axis_index("s")
    wid = c * 16 + s
    def _z(i, carry):
        zbuf[pl.ds(i * 16, 16)] = jnp.zeros((16,), jnp.float32)
        return carry
    lax.fori_loop(0, _RPT // 16, _z, 0)
    pltpu.sync_copy(zbuf, degacc.at[pl.ds(s * _RPT, _RPT)])
    plsc.subcore_barrier()

    def _dgroup(g, carry):
        sh = s + 16 * (g % 2)
        gg = g // 2
        pltpu.sync_copy(col_hbm.at[sh, pl.ds(gg * _GRP, _GRP)], colv)
        pltpu.sync_copy(ew_hbm.at[sh, pl.ds(gg * _GRP, _GRP)], ewv)
        def _dchunk(i, c1):
            pltpu.sync_copy(ewv.at[i], degacc.at[colv.at[i]], add=True)
            return c1
        lax.fori_loop(0, _GRP, _dchunk, 0)
        return carry
    lax.fori_loop(0, 2 * _NGRP, _dgroup, 0)
    plsc.subcore_barrier()

    pltpu.sync_copy(degacc, disv)
    def _rsq(k, carry):
        sl = pl.ds(k * 16, 16)
        xv = disv[sl] + 1.0
        iv = 0x5F3759DF - lax.shift_right_logical(plsc.bitcast(xv, jnp.int32), 1)
        y = plsc.bitcast(iv, jnp.float32)
        xh = 0.5 * xv
        for _ in range(3):
            y = y * (1.5 - xh * y * y)
        disv[sl] = y
        return carry
    lax.fori_loop(0, _NPAD // 16, _rsq, 0)

    def _ngroup(g, carry):
        pltpu.sync_copy(row_hbm.at[wid, pl.ds(g * _GRP, _GRP)], rowv)
        pltpu.sync_copy(col_hbm.at[wid, pl.ds(g * _GRP, _GRP)], colv)
        pltpu.sync_copy(ew_hbm.at[wid, pl.ds(g * _GRP, _GRP)], ewv)
        def _nchunk(i, c1):
            for eb in range(_CH // 16):
                sl = pl.ds(eb * 16, 16)
                r16 = rowv[i, sl]
                c16 = colv[i, sl]
                e16 = ewv[i, sl]
                dr = plsc.load_gather(disv, [r16])
                dc = plsc.load_gather(disv, [c16])
                normb[i, sl] = dr * e16 * dc
            return c1
        lax.fori_loop(0, _GRP, _nchunk, 0)
        pltpu.sync_copy(normb, norm_hbm.at[wid, pl.ds(g * _GRP, _GRP)])
        return carry
    lax.fori_loop(0, _NGRP, _ngroup, 0)
    @pl.when(jnp.logical_and(c == 0, s == 0))
    def _():
        pltpu.sync_copy(disv, dis_hbm)


def _norm_sc(row_p, col_p, ew_p):
    mesh = plsc.VectorSubcoreMesh(core_axis_name="c", subcore_axis_name="s")
    f = pl.kernel(
        _norm_kernel,
        out_type=(jax.ShapeDtypeStruct((_NSHARD, _NCHUNK, _CH), jnp.float32),
                  jax.ShapeDtypeStruct((_NPAD,), jnp.float32)),
        mesh=mesh,
        scratch_types=[
            pltpu.VMEM((_GRP, _CH), jnp.int32),
            pltpu.VMEM((_GRP, _CH), jnp.int32),
            pltpu.VMEM((_GRP, _CH), jnp.float32),
            pltpu.VMEM((_GRP, _CH), jnp.float32),
            pltpu.VMEM((_RPT,), jnp.float32),
            pltpu.VMEM((_NPAD,), jnp.float32),
            pltpu.VMEM_SHARED((_NPAD,), jnp.float32),
            pltpu.SemaphoreType.DMA,
        ],
        compiler_params=pltpu.CompilerParams(needs_layout_passes=False),
    )
    return f(row_p, col_p, ew_p)


def _combine_body(p_ref, h_ref, h0_ref, dis_ref, w_ref, g_ref, be_ref, out_ref):
    inv_deg = dis_ref[...] * dis_ref[...]
    agg = p_ref[0, :_N, :] + p_ref[1, :_N, :] + h_ref[...] * inv_deg
    hcomb = (1.0 - _ALPHA) * agg + _ALPHA * h0_ref[...]
    hc = jnp.dot(hcomb, w_ref[...], preferred_element_type=jnp.float32)
    hc = jnp.maximum(hc, 0.0)
    m = jnp.mean(hc, axis=0, keepdims=True)
    v = jnp.mean((hc - m) ** 2, axis=0, keepdims=True)
    out_ref[...] = g_ref[...] * (hc - m) / jnp.sqrt(v + 1e-5) + be_ref[...]


def _combine(p, h, h0, dis, w, g, be):
    return pl.pallas_call(
        _combine_body,
        out_shape=jax.ShapeDtypeStruct((_N, _H), jnp.float32),
    )(p, h, h0, dis.reshape(_N, 1), w, g.reshape(1, _H), be.reshape(1, _H))


def _bn(x, g, b, eps=1e-5):
    m = jnp.mean(x, axis=0)
    v = jnp.var(x, axis=0)
    return g * (x - m) / jnp.sqrt(v + eps) + b


def _mlp_block(x, W, b, g, be):
    x1 = x @ W + b
    x2 = jax.nn.relu(x1)
    x2 = _bn(x2, g, be)
    return x2 + x1


def kernel(x, edge_attr, x_10d, lin_first, gcn_params, ewmlp_params, head_params, edge_index, batch):
    ew = edge_attr
    for (W, b, g, be) in ewmlp_params:
        ew = _mlp_block(ew, W, b, g, be)
    ew = jax.nn.sigmoid(ew)[:, 0]
    Wf, bf = lin_first
    h = x @ Wf + bf
    h0 = h
    pad = ((0, 0), (0, _EPT - 640000 // _NSHARD))
    row_p = jnp.pad(edge_index[0].reshape(_NSHARD, -1), pad).reshape(_NSHARD, _NCHUNK, _CH)
    col_p = jnp.pad(edge_index[1].reshape(_NSHARD, -1), pad).reshape(_NSHARD, _NCHUNK, _CH)
    ew_p = jnp.pad(ew.reshape(_NSHARD, -1), pad).reshape(_NSHARD, _NCHUNK, _CH)
    norm_p, dis_pad = _norm_sc(row_p, col_p, ew_p)
    dis = dis_pad[:_N]
    for (W1, g, be) in gcn_params:
        p = _spmm(h, row_p, col_p, norm_p)
        h = _combine(p, h, h0, dis, W1, g, be)
    x_aggr = jax.ops.segment_sum(h, batch, num_segments=_NG)
    x_aggr = jax.nn.sigmoid(x_aggr)
    x_aggr = jnp.concatenate([x_aggr, x_10d], axis=1)
    out = x_aggr
    for (W, b, g, be) in head_params:
        out = _mlp_block(out, W, b, g, be)
    out = jax.nn.sigmoid(out)
    return (out, x_aggr)
